# baseline (speedup 1.0000x reference)
import functools

import jax
import jax.numpy as jnp
from jax import lax
from jax.experimental import pallas as pl
from jax.experimental.pallas import tpu as pltpu
from jax.experimental.pallas import tpu_sc as plsc

_N = 10000
_D = 128
_NPAD = 10224
_NW = 32
_CH = 128
_NCH = 80
_EPAD = _NW * _NCH * _CH
_WB = 80
_NPADD = 10240
_NPTD = _NPADD // 16


def _sc_degree(dsts):
    mesh = plsc.VectorSubcoreMesh(core_axis_name="c", subcore_axis_name="s")

    @functools.partial(
        pl.kernel,
        out_type=jax.ShapeDtypeStruct((2, _NPADD), jnp.float32),
        mesh=mesh,
        scratch_types=[
            pltpu.VMEM((_NCH, _CH), jnp.int32),
            pltpu.VMEM((_CH,), jnp.float32),
            pltpu.VMEM((_NPTD,), jnp.float32),
            pltpu.VMEM_SHARED((_NPADD,), jnp.float32),
        ],
    )
    def k(dsts_hbm, out_hbm, dst_v, ones_v, stage_v, deg_sh):
        c = lax.axis_index("c")
        s = lax.axis_index("s")
        wid = s * 2 + c
        for i in range(_CH // 16):
            ones_v[pl.ds(i * 16, 16)] = jnp.ones((16,), jnp.float32)
        for i in range(_NPTD // 16):
            stage_v[pl.ds(i * 16, 16)] = jnp.zeros((16,), jnp.float32)
        pltpu.sync_copy(stage_v, deg_sh.at[pl.ds(s * _NPTD, _NPTD)])
        plsc.subcore_barrier()
        pltpu.sync_copy(dsts_hbm.at[wid], dst_v)

        def chunk(j, carry):
            pltpu.sync_copy(ones_v, deg_sh.at[dst_v.at[j]], add=True)
            return carry

        lax.fori_loop(0, _NCH, chunk, 0)
        plsc.subcore_barrier()
        pltpu.sync_copy(deg_sh.at[pl.ds(s * _NPTD, _NPTD)], stage_v)
        pltpu.sync_copy(stage_v, out_hbm.at[c, pl.ds(s * _NPTD, _NPTD)])

    return k(dsts)


def _sc_edge_agg(hp, srcs, dsts):
    mesh = plsc.VectorSubcoreMesh(core_axis_name="c", subcore_axis_name="s")

    @functools.partial(
        pl.kernel,
        out_type=jax.ShapeDtypeStruct((2, _NPAD, _D), jnp.float32),
        mesh=mesh,
        scratch_types=[
            pltpu.VMEM((_NCH, _CH), jnp.int32),
            pltpu.VMEM((_NCH, _CH), jnp.int32),
            pltpu.VMEM((_CH, _D), jnp.float32),
            pltpu.VMEM_SHARED((_NPAD, _D), jnp.float32),
            pltpu.SemaphoreType.DMA,
        ],
    )
    def k(hp_hbm, srcs_hbm, dsts_hbm, out_hbm, src_v, dst_v, rows_a,
          acc_sh, sem_a):
        c = lax.axis_index("c")
        s = lax.axis_index("s")
        wid = s * 2 + c

        def zrow(i, carry):
            for kk in range(_D // 16):
                rows_a[i, pl.ds(kk * 16, 16)] = jnp.zeros((16,), jnp.float32)
            return carry

        lax.fori_loop(0, _WB, zrow, 0)

        @pl.when(s < 15)
        def _():
            for kk in range(8):
                pltpu.sync_copy(rows_a.at[pl.ds(0, _WB)],
                                acc_sh.at[pl.ds(s * 640 + kk * _WB, _WB)])

        @pl.when(s == 15)
        def _():
            for kk in range(7):
                pltpu.sync_copy(rows_a.at[pl.ds(0, _WB)],
                                acc_sh.at[pl.ds(9600 + kk * _WB, _WB)])
            pltpu.sync_copy(rows_a.at[pl.ds(0, 64)],
                            acc_sh.at[pl.ds(10160, 64)])

        plsc.subcore_barrier()

        pltpu.sync_copy(srcs_hbm.at[wid], src_v)
        pltpu.sync_copy(dsts_hbm.at[wid], dst_v)

        def chunk(j, carry):
            pltpu.async_copy(hp_hbm.at[src_v.at[j]], rows_a, sem_a).wait()
            pltpu.sync_copy(rows_a, acc_sh.at[dst_v.at[j]], add=True)
            return carry

        lax.fori_loop(0, _NCH, chunk, 0)
        plsc.subcore_barrier()

        @pl.when(s < 15)
        def _():
            for kk in range(8):
                sl = pl.ds(s * 640 + kk * _WB, _WB)
                pltpu.sync_copy(acc_sh.at[sl], rows_a.at[pl.ds(0, _WB)])
                pltpu.sync_copy(rows_a.at[pl.ds(0, _WB)], out_hbm.at[c, sl])

        @pl.when(s == 15)
        def _():
            for kk in range(7):
                sl = pl.ds(9600 + kk * _WB, _WB)
                pltpu.sync_copy(acc_sh.at[sl], rows_a.at[pl.ds(0, _WB)])
                pltpu.sync_copy(rows_a.at[pl.ds(0, _WB)], out_hbm.at[c, sl])
            sl = pl.ds(10160, 64)
            pltpu.sync_copy(acc_sh.at[sl], rows_a.at[pl.ds(0, 64)])
            pltpu.sync_copy(rows_a.at[pl.ds(0, 64)], out_hbm.at[c, sl])

    return k(hp, srcs, dsts)


_BR = 1704


def _gelu(x):
    return 0.5 * x * (1.0 + lax.erf(x * (2.0 ** -0.5)))


def _t1_body(x_ref, deg_ref, w_ref, dis_ref, hp_ref):
    deg = deg_ref[0] + deg_ref[1] + 1.0
    dis = lax.rsqrt(deg)
    dis_ref[...] = dis
    hp_ref[...] = dis * jnp.dot(x_ref[...], w_ref[...],
                                preferred_element_type=jnp.float32)


def _tc_first(xp, degs, w1):
    return pl.pallas_call(
        _t1_body,
        grid=(_NPAD // _BR,),
        in_specs=[
            pl.BlockSpec((_BR, _D), lambda i: (i, 0)),
            pl.BlockSpec((2, _BR, 1), lambda i: (0, i, 0)),
            pl.BlockSpec((_D, _D), lambda i: (0, 0)),
        ],
        out_specs=[
            pl.BlockSpec((_BR, 1), lambda i: (i, 0)),
            pl.BlockSpec((_BR, _D), lambda i: (i, 0)),
        ],
        out_shape=[
            jax.ShapeDtypeStruct((_NPAD, 1), jnp.float32),
            jax.ShapeDtypeStruct((_NPAD, _D), jnp.float32),
        ],
    )(xp, degs, w1)


def _tmid_body(acc_ref, hp_ref, dis_ref, b_ref, w_ref, out_ref):
    ssum = acc_ref[0] + acc_ref[1] + hp_ref[...]
    dis = dis_ref[...]
    pre = dis * ssum + b_ref[...]
    xg = _gelu(pre)
    out_ref[...] = dis * jnp.dot(xg, w_ref[...],
                                 preferred_element_type=jnp.float32)


def _tc_mid(acc, hp, dis, b, w):
    return pl.pallas_call(
        _tmid_body,
        grid=(_NPAD // _BR,),
        in_specs=[
            pl.BlockSpec((2, _BR, _D), lambda i: (0, i, 0)),
            pl.BlockSpec((_BR, _D), lambda i: (i, 0)),
            pl.BlockSpec((_BR, 1), lambda i: (i, 0)),
            pl.BlockSpec((1, _D), lambda i: (0, 0)),
            pl.BlockSpec((_D, _D), lambda i: (0, 0)),
        ],
        out_specs=pl.BlockSpec((_BR, _D), lambda i: (i, 0)),
        out_shape=jax.ShapeDtypeStruct((_NPAD, _D), jnp.float32),
    )(acc, hp, dis, b, w)


def _tfin_body(acc_ref, hp_ref, dis_ref, b_ref, out_ref):
    ssum = acc_ref[0] + acc_ref[1] + hp_ref[...]
    out_ref[...] = dis_ref[...] * ssum + b_ref[...]


def _tc_final(acc, hp, dis, b):
    return pl.pallas_call(
        _tfin_body,
        grid=(_NPAD // _BR,),
        in_specs=[
            pl.BlockSpec((2, _BR, _D), lambda i: (0, i, 0)),
            pl.BlockSpec((_BR, _D), lambda i: (i, 0)),
            pl.BlockSpec((_BR, 1), lambda i: (i, 0)),
            pl.BlockSpec((1, _D), lambda i: (0, 0)),
        ],
        out_specs=pl.BlockSpec((_BR, _D), lambda i: (i, 0)),
        out_shape=jax.ShapeDtypeStruct((_NPAD, _D), jnp.float32),
    )(acc, hp, dis, b)


def kernel(x_piece, edge_index_piece, batch, W1, b1, W2, b2, W3, b3):
    del batch
    src = edge_index_piece[0].astype(jnp.int32)
    dst = edge_index_piece[1].astype(jnp.int32)
    e = src.shape[0]
    pad = _EPAD - e
    srcp = jnp.concatenate([src, jnp.full((pad,), _N, jnp.int32)])
    dstp = jnp.concatenate([dst, jnp.full((pad,), _N, jnp.int32)])
    srcp = srcp.reshape(_NW, _NCH, _CH)
    dstp = dstp.reshape(_NW, _NCH, _CH)
    xp = jnp.concatenate(
        [x_piece, jnp.zeros((_NPAD - _N, _D), jnp.float32)], axis=0)

    degs = _sc_degree(dstp)[:, :_NPAD].reshape(2, _NPAD, 1)
    dis, hp1 = _tc_first(xp, degs, W1)
    acc1 = _sc_edge_agg(hp1, srcp, dstp)
    hp2 = _tc_mid(acc1, hp1, dis, b1.reshape(1, _D), W2)
    acc2 = _sc_edge_agg(hp2, srcp, dstp)
    hp3 = _tc_mid(acc2, hp2, dis, b2.reshape(1, _D), W3)
    acc3 = _sc_edge_agg(hp3, srcp, dstp)
    out = _tc_final(acc3, hp3, dis, b3.reshape(1, _D))
    return out[:_N]


# speedup vs baseline: 2.6723x; 2.6723x over previous
import functools

import jax
import jax.numpy as jnp
from jax import lax
from jax.experimental import pallas as pl
from jax.experimental.pallas import tpu as pltpu
from jax.experimental.pallas import tpu_sc as plsc

_N = 10000
_D = 128
_NPAD = 10224
_NW = 32
_CH = 128
_NCH = 79
_EPAD = _NW * _NCH * _CH
_WB = 80
_NPADD = 10240
_NPTD = _NPADD // 16


def _sc_degree(dsts):
    mesh = plsc.VectorSubcoreMesh(core_axis_name="c", subcore_axis_name="s")

    @functools.partial(
        pl.kernel,
        out_type=jax.ShapeDtypeStruct((2, _NPADD), jnp.float32),
        mesh=mesh,
        scratch_types=[
            pltpu.VMEM((_NCH, _CH), jnp.int32),
            pltpu.VMEM((_CH,), jnp.float32),
            pltpu.VMEM((_NPTD,), jnp.float32),
            pltpu.VMEM_SHARED((_NPADD,), jnp.float32),
        ],
    )
    def k(dsts_hbm, out_hbm, dst_v, ones_v, stage_v, deg_sh):
        c = lax.axis_index("c")
        s = lax.axis_index("s")
        wid = s * 2 + c
        for i in range(_CH // 16):
            ones_v[pl.ds(i * 16, 16)] = jnp.ones((16,), jnp.float32)
        for i in range(_NPTD // 16):
            stage_v[pl.ds(i * 16, 16)] = jnp.zeros((16,), jnp.float32)
        pltpu.sync_copy(stage_v, deg_sh.at[pl.ds(s * _NPTD, _NPTD)])
        plsc.subcore_barrier()
        pltpu.sync_copy(dsts_hbm.at[wid], dst_v)

        def chunk(j, carry):
            pltpu.sync_copy(ones_v, deg_sh.at[dst_v.at[j]], add=True)
            return carry

        lax.fori_loop(0, _NCH, chunk, 0)
        plsc.subcore_barrier()
        pltpu.sync_copy(deg_sh.at[pl.ds(s * _NPTD, _NPTD)], stage_v)
        pltpu.sync_copy(stage_v, out_hbm.at[c, pl.ds(s * _NPTD, _NPTD)])

    return k(dsts)


def _sc_edge_agg(hp, srcs, dsts):
    mesh = plsc.VectorSubcoreMesh(core_axis_name="c", subcore_axis_name="s")

    @functools.partial(
        pl.kernel,
        out_type=jax.ShapeDtypeStruct((2, _NPAD, _D), jnp.float32),
        mesh=mesh,
        scratch_types=[
            pltpu.VMEM((_NCH, _CH), jnp.int32),
            pltpu.VMEM((_NCH, _CH), jnp.int32),
            pltpu.VMEM((_CH, _D), jnp.float32),
            pltpu.VMEM_SHARED((_NPAD, _D), jnp.float32),
            pltpu.SemaphoreType.DMA,
        ],
    )
    def k(hp_hbm, srcs_hbm, dsts_hbm, out_hbm, src_v, dst_v, rows_a,
          acc_sh, sem_a):
        c = lax.axis_index("c")
        s = lax.axis_index("s")
        wid = s * 2 + c

        def zrow(i, carry):
            for kk in range(_D // 16):
                rows_a[i, pl.ds(kk * 16, 16)] = jnp.zeros((16,), jnp.float32)
            return carry

        lax.fori_loop(0, _WB, zrow, 0)

        @pl.when(s < 15)
        def _():
            for kk in range(8):
                pltpu.sync_copy(rows_a.at[pl.ds(0, _WB)],
                                acc_sh.at[pl.ds(s * 640 + kk * _WB, _WB)])

        @pl.when(s == 15)
        def _():
            for kk in range(7):
                pltpu.sync_copy(rows_a.at[pl.ds(0, _WB)],
                                acc_sh.at[pl.ds(9600 + kk * _WB, _WB)])
            pltpu.sync_copy(rows_a.at[pl.ds(0, 64)],
                            acc_sh.at[pl.ds(10160, 64)])

        plsc.subcore_barrier()

        pltpu.sync_copy(srcs_hbm.at[wid], src_v)
        pltpu.sync_copy(dsts_hbm.at[wid], dst_v)

        def chunk(j, carry):
            pltpu.async_copy(hp_hbm.at[src_v.at[j]], rows_a, sem_a).wait()
            pltpu.sync_copy(rows_a, acc_sh.at[dst_v.at[j]], add=True)
            return carry

        lax.fori_loop(0, _NCH, chunk, 0)
        plsc.subcore_barrier()

        @pl.when(s < 15)
        def _():
            for kk in range(8):
                sl = pl.ds(s * 640 + kk * _WB, _WB)
                pltpu.sync_copy(acc_sh.at[sl], rows_a.at[pl.ds(0, _WB)])
                pltpu.sync_copy(rows_a.at[pl.ds(0, _WB)], out_hbm.at[c, sl])

        @pl.when(s == 15)
        def _():
            for kk in range(7):
                sl = pl.ds(9600 + kk * _WB, _WB)
                pltpu.sync_copy(acc_sh.at[sl], rows_a.at[pl.ds(0, _WB)])
                pltpu.sync_copy(rows_a.at[pl.ds(0, _WB)], out_hbm.at[c, sl])
            sl = pl.ds(10160, 64)
            pltpu.sync_copy(acc_sh.at[sl], rows_a.at[pl.ds(0, 64)])
            pltpu.sync_copy(rows_a.at[pl.ds(0, 64)], out_hbm.at[c, sl])

    return k(hp, srcs, dsts)


_BR = 1704


def _gelu(x):
    return 0.5 * x * (1.0 + lax.erf(x * (2.0 ** -0.5)))


def _t1_body(x_ref, deg_ref, w_ref, dis_ref, hp_ref):
    deg = deg_ref[0] + deg_ref[1] + 1.0
    dis = lax.rsqrt(deg)
    dis_ref[...] = dis
    hp_ref[...] = dis * jnp.dot(x_ref[...], w_ref[...],
                                preferred_element_type=jnp.float32)


def _tc_first(xp, degs, w1):
    return pl.pallas_call(
        _t1_body,
        grid=(_NPAD // _BR,),
        in_specs=[
            pl.BlockSpec((_BR, _D), lambda i: (i, 0)),
            pl.BlockSpec((2, _BR, 1), lambda i: (0, i, 0)),
            pl.BlockSpec((_D, _D), lambda i: (0, 0)),
        ],
        out_specs=[
            pl.BlockSpec((_BR, 1), lambda i: (i, 0)),
            pl.BlockSpec((_BR, _D), lambda i: (i, 0)),
        ],
        out_shape=[
            jax.ShapeDtypeStruct((_NPAD, 1), jnp.float32),
            jax.ShapeDtypeStruct((_NPAD, _D), jnp.float32),
        ],
    )(xp, degs, w1)


def _tmid_body(acc_ref, hp_ref, dis_ref, b_ref, w_ref, out_ref):
    ssum = acc_ref[0] + acc_ref[1] + hp_ref[...]
    dis = dis_ref[...]
    pre = dis * ssum + b_ref[...]
    xg = _gelu(pre)
    out_ref[...] = dis * jnp.dot(xg, w_ref[...],
                                 preferred_element_type=jnp.float32)


def _tc_mid(acc, hp, dis, b, w):
    return pl.pallas_call(
        _tmid_body,
        grid=(_NPAD // _BR,),
        in_specs=[
            pl.BlockSpec((2, _BR, _D), lambda i: (0, i, 0)),
            pl.BlockSpec((_BR, _D), lambda i: (i, 0)),
            pl.BlockSpec((_BR, 1), lambda i: (i, 0)),
            pl.BlockSpec((1, _D), lambda i: (0, 0)),
            pl.BlockSpec((_D, _D), lambda i: (0, 0)),
        ],
        out_specs=pl.BlockSpec((_BR, _D), lambda i: (i, 0)),
        out_shape=jax.ShapeDtypeStruct((_NPAD, _D), jnp.float32),
    )(acc, hp, dis, b, w)


def _tfin_body(acc_ref, hp_ref, dis_ref, b_ref, out_ref):
    ssum = acc_ref[0] + acc_ref[1] + hp_ref[...]
    out_ref[...] = dis_ref[...] * ssum + b_ref[...]


def _tc_final(acc, hp, dis, b):
    return pl.pallas_call(
        _tfin_body,
        grid=(_NPAD // _BR,),
        in_specs=[
            pl.BlockSpec((2, _BR, _D), lambda i: (0, i, 0)),
            pl.BlockSpec((_BR, _D), lambda i: (i, 0)),
            pl.BlockSpec((_BR, 1), lambda i: (i, 0)),
            pl.BlockSpec((1, _D), lambda i: (0, 0)),
        ],
        out_specs=pl.BlockSpec((_BR, _D), lambda i: (i, 0)),
        out_shape=jax.ShapeDtypeStruct((_NPAD, _D), jnp.float32),
    )(acc, hp, dis, b)


def kernel(x_piece, edge_index_piece, batch, W1, b1, W2, b2, W3, b3):
    del batch
    src = edge_index_piece[0].astype(jnp.int32)
    dst = edge_index_piece[1].astype(jnp.int32)
    e = src.shape[0]
    pad = _EPAD - e
    padidx = _N + (jnp.arange(pad, dtype=jnp.int32) % (_NPAD - _N))
    srcp = jnp.concatenate([src, padidx])
    dstp = jnp.concatenate([dst, padidx])
    srcp = srcp.reshape(_NW, _NCH, _CH)
    dstp = dstp.reshape(_NW, _NCH, _CH)
    xp = jnp.concatenate(
        [x_piece, jnp.zeros((_NPAD - _N, _D), jnp.float32)], axis=0)

    degs = _sc_degree(dstp)[:, :_NPAD].reshape(2, _NPAD, 1)
    dis, hp1 = _tc_first(xp, degs, W1)
    acc1 = _sc_edge_agg(hp1, srcp, dstp)
    hp2 = _tc_mid(acc1, hp1, dis, b1.reshape(1, _D), W2)
    acc2 = _sc_edge_agg(hp2, srcp, dstp)
    hp3 = _tc_mid(acc2, hp2, dis, b2.reshape(1, _D), W3)
    acc3 = _sc_edge_agg(hp3, srcp, dstp)
    out = _tc_final(acc3, hp3, dis, b3.reshape(1, _D))
    return out[:_N]


# speedup vs baseline: 3.3833x; 1.2661x over previous
import functools

import jax
import jax.numpy as jnp
from jax import lax
from jax.experimental import pallas as pl
from jax.experimental.pallas import tpu as pltpu
from jax.experimental.pallas import tpu_sc as plsc

_N = 10000
_D = 128
_NPAD = 10224
_NW = 32
_CH = 128
_NCH = 80
_EPAD = _NW * _NCH * _CH
_WB = 80
_NPADD = 10240
_NPTD = _NPADD // 16


def _sc_degree(dsts):
    mesh = plsc.VectorSubcoreMesh(core_axis_name="c", subcore_axis_name="s")

    @functools.partial(
        pl.kernel,
        out_type=jax.ShapeDtypeStruct((2, _NPADD), jnp.float32),
        mesh=mesh,
        scratch_types=[
            pltpu.VMEM((_NCH, _CH), jnp.int32),
            pltpu.VMEM((_CH,), jnp.float32),
            pltpu.VMEM((_NPTD,), jnp.float32),
            pltpu.VMEM_SHARED((_NPADD,), jnp.float32),
        ],
    )
    def k(dsts_hbm, out_hbm, dst_v, ones_v, stage_v, deg_sh):
        c = lax.axis_index("c")
        s = lax.axis_index("s")
        wid = s * 2 + c
        for i in range(_CH // 16):
            ones_v[pl.ds(i * 16, 16)] = jnp.ones((16,), jnp.float32)
        for i in range(_NPTD // 16):
            stage_v[pl.ds(i * 16, 16)] = jnp.zeros((16,), jnp.float32)
        pltpu.sync_copy(stage_v, deg_sh.at[pl.ds(s * _NPTD, _NPTD)])
        plsc.subcore_barrier()
        pltpu.sync_copy(dsts_hbm.at[wid], dst_v)

        def chunk(j, carry):
            pltpu.sync_copy(ones_v, deg_sh.at[dst_v.at[j]], add=True)
            return carry

        lax.fori_loop(0, _NCH, chunk, 0)
        plsc.subcore_barrier()
        pltpu.sync_copy(deg_sh.at[pl.ds(s * _NPTD, _NPTD)], stage_v)
        pltpu.sync_copy(stage_v, out_hbm.at[c, pl.ds(s * _NPTD, _NPTD)])

    return k(dsts)


def _sc_edge_agg(hp, srcs, dsts):
    mesh = plsc.VectorSubcoreMesh(core_axis_name="c", subcore_axis_name="s")

    @functools.partial(
        pl.kernel,
        out_type=jax.ShapeDtypeStruct((2, _NPAD, _D), jnp.float32),
        mesh=mesh,
        scratch_types=[
            pltpu.VMEM((_NCH // 2, _CH), jnp.int32),
            pltpu.VMEM((_NCH // 2, _CH), jnp.int32),
            pltpu.VMEM((_CH, _D), jnp.float32),
            pltpu.VMEM((_CH, _D), jnp.float32),
            pltpu.VMEM_SHARED((_NPAD, _D), jnp.float32),
            pltpu.SemaphoreType.DMA,
            pltpu.SemaphoreType.DMA,
            pltpu.SemaphoreType.DMA,
            pltpu.SemaphoreType.DMA,
        ],
    )
    def k(hp_hbm, srcs_hbm, dsts_hbm, out_hbm, src_v, dst_v, rows_a, rows_b,
          acc_sh, sem_ga, sem_gb, sem_sa, sem_sb):
        c = lax.axis_index("c")
        s = lax.axis_index("s")
        wid = s * 2 + c

        def zrow(i, carry):
            for kk in range(_D // 16):
                rows_a[i, pl.ds(kk * 16, 16)] = jnp.zeros((16,), jnp.float32)
            return carry

        lax.fori_loop(0, _WB, zrow, 0)

        @pl.when(s < 15)
        def _():
            for kk in range(8):
                pltpu.sync_copy(rows_a.at[pl.ds(0, _WB)],
                                acc_sh.at[pl.ds(s * 640 + kk * _WB, _WB)])

        @pl.when(s == 15)
        def _():
            for kk in range(7):
                pltpu.sync_copy(rows_a.at[pl.ds(0, _WB)],
                                acc_sh.at[pl.ds(9600 + kk * _WB, _WB)])
            pltpu.sync_copy(rows_a.at[pl.ds(0, 64)],
                            acc_sh.at[pl.ds(10160, 64)])

        plsc.subcore_barrier()

        half = _NCH // 2
        for ph in range(2):
            pltpu.sync_copy(srcs_hbm.at[wid, pl.ds(ph * half, half)], src_v)
            pltpu.sync_copy(dsts_hbm.at[wid, pl.ds(ph * half, half)], dst_v)
            pltpu.async_copy(hp_hbm.at[src_v.at[0]], rows_a, sem_ga)

            def chunk2(t, carry):
                j0 = 2 * t
                pltpu.make_async_copy(hp_hbm.at[src_v.at[j0]], rows_a,
                                      sem_ga).wait()
                pltpu.async_copy(rows_a, acc_sh.at[dst_v.at[j0]], sem_sa,
                                 add=True)

                @pl.when(t > 0)
                def _():
                    pltpu.make_async_copy(rows_b, acc_sh.at[dst_v.at[j0 - 1]],
                                          sem_sb).wait()

                pltpu.async_copy(hp_hbm.at[src_v.at[j0 + 1]], rows_b, sem_gb)
                pltpu.make_async_copy(hp_hbm.at[src_v.at[j0 + 1]], rows_b,
                                      sem_gb).wait()
                pltpu.async_copy(rows_b, acc_sh.at[dst_v.at[j0 + 1]], sem_sb,
                                 add=True)
                pltpu.make_async_copy(rows_a, acc_sh.at[dst_v.at[j0]],
                                      sem_sa).wait()

                @pl.when(t < half // 2 - 1)
                def _():
                    pltpu.async_copy(hp_hbm.at[src_v.at[j0 + 2]], rows_a,
                                     sem_ga)

                return carry

            lax.fori_loop(0, half // 2, chunk2, 0)
            pltpu.make_async_copy(rows_b, acc_sh.at[dst_v.at[half - 1]],
                                  sem_sb).wait()
        plsc.subcore_barrier()

        @pl.when(s < 15)
        def _():
            for kk in range(8):
                sl = pl.ds(s * 640 + kk * _WB, _WB)
                pltpu.sync_copy(acc_sh.at[sl], rows_a.at[pl.ds(0, _WB)])
                pltpu.sync_copy(rows_a.at[pl.ds(0, _WB)], out_hbm.at[c, sl])

        @pl.when(s == 15)
        def _():
            for kk in range(7):
                sl = pl.ds(9600 + kk * _WB, _WB)
                pltpu.sync_copy(acc_sh.at[sl], rows_a.at[pl.ds(0, _WB)])
                pltpu.sync_copy(rows_a.at[pl.ds(0, _WB)], out_hbm.at[c, sl])
            sl = pl.ds(10160, 64)
            pltpu.sync_copy(acc_sh.at[sl], rows_a.at[pl.ds(0, 64)])
            pltpu.sync_copy(rows_a.at[pl.ds(0, 64)], out_hbm.at[c, sl])

    return k(hp, srcs, dsts)


_BR = 1704


def _gelu(x):
    return 0.5 * x * (1.0 + lax.erf(x * (2.0 ** -0.5)))


def _t1_body(x_ref, deg_ref, w_ref, dis_ref, hp_ref):
    deg = deg_ref[0] + deg_ref[1] + 1.0
    dis = lax.rsqrt(deg)
    dis_ref[...] = dis
    hp_ref[...] = dis * jnp.dot(x_ref[...], w_ref[...],
                                preferred_element_type=jnp.float32)


def _tc_first(xp, degs, w1):
    return pl.pallas_call(
        _t1_body,
        grid=(_NPAD // _BR,),
        in_specs=[
            pl.BlockSpec((_BR, _D), lambda i: (i, 0)),
            pl.BlockSpec((2, _BR, 1), lambda i: (0, i, 0)),
            pl.BlockSpec((_D, _D), lambda i: (0, 0)),
        ],
        out_specs=[
            pl.BlockSpec((_BR, 1), lambda i: (i, 0)),
            pl.BlockSpec((_BR, _D), lambda i: (i, 0)),
        ],
        out_shape=[
            jax.ShapeDtypeStruct((_NPAD, 1), jnp.float32),
            jax.ShapeDtypeStruct((_NPAD, _D), jnp.float32),
        ],
    )(xp, degs, w1)


def _tmid_body(acc_ref, hp_ref, dis_ref, b_ref, w_ref, out_ref):
    ssum = acc_ref[0] + acc_ref[1] + hp_ref[...]
    dis = dis_ref[...]
    pre = dis * ssum + b_ref[...]
    xg = _gelu(pre)
    out_ref[...] = dis * jnp.dot(xg, w_ref[...],
                                 preferred_element_type=jnp.float32)


def _tc_mid(acc, hp, dis, b, w):
    return pl.pallas_call(
        _tmid_body,
        grid=(_NPAD // _BR,),
        in_specs=[
            pl.BlockSpec((2, _BR, _D), lambda i: (0, i, 0)),
            pl.BlockSpec((_BR, _D), lambda i: (i, 0)),
            pl.BlockSpec((_BR, 1), lambda i: (i, 0)),
            pl.BlockSpec((1, _D), lambda i: (0, 0)),
            pl.BlockSpec((_D, _D), lambda i: (0, 0)),
        ],
        out_specs=pl.BlockSpec((_BR, _D), lambda i: (i, 0)),
        out_shape=jax.ShapeDtypeStruct((_NPAD, _D), jnp.float32),
    )(acc, hp, dis, b, w)


def _tfin_body(acc_ref, hp_ref, dis_ref, b_ref, out_ref):
    ssum = acc_ref[0] + acc_ref[1] + hp_ref[...]
    out_ref[...] = dis_ref[...] * ssum + b_ref[...]


def _tc_final(acc, hp, dis, b):
    return pl.pallas_call(
        _tfin_body,
        grid=(_NPAD // _BR,),
        in_specs=[
            pl.BlockSpec((2, _BR, _D), lambda i: (0, i, 0)),
            pl.BlockSpec((_BR, _D), lambda i: (i, 0)),
            pl.BlockSpec((_BR, 1), lambda i: (i, 0)),
            pl.BlockSpec((1, _D), lambda i: (0, 0)),
        ],
        out_specs=pl.BlockSpec((_BR, _D), lambda i: (i, 0)),
        out_shape=jax.ShapeDtypeStruct((_NPAD, _D), jnp.float32),
    )(acc, hp, dis, b)


def kernel(x_piece, edge_index_piece, batch, W1, b1, W2, b2, W3, b3):
    del batch
    src = edge_index_piece[0].astype(jnp.int32)
    dst = edge_index_piece[1].astype(jnp.int32)
    e = src.shape[0]
    pad = _EPAD - e
    padidx = _N + (jnp.arange(pad, dtype=jnp.int32) % (_NPAD - _N))
    srcp = jnp.concatenate([src, padidx])
    dstp = jnp.concatenate([dst, padidx])
    srcp = srcp.reshape(_NW, _NCH, _CH)
    dstp = dstp.reshape(_NW, _NCH, _CH)
    xp = jnp.concatenate(
        [x_piece, jnp.zeros((_NPAD - _N, _D), jnp.float32)], axis=0)

    degs = _sc_degree(dstp)[:, :_NPAD].reshape(2, _NPAD, 1)
    dis, hp1 = _tc_first(xp, degs, W1)
    acc1 = _sc_edge_agg(hp1, srcp, dstp)
    hp2 = _tc_mid(acc1, hp1, dis, b1.reshape(1, _D), W2)
    acc2 = _sc_edge_agg(hp2, srcp, dstp)
    hp3 = _tc_mid(acc2, hp2, dis, b2.reshape(1, _D), W3)
    acc3 = _sc_edge_agg(hp3, srcp, dstp)
    out = _tc_final(acc3, hp3, dis, b3.reshape(1, _D))
    return out[:_N]


# speedup vs baseline: 3.3912x; 1.0023x over previous
import functools

import jax
import jax.numpy as jnp
from jax import lax
from jax.experimental import pallas as pl
from jax.experimental.pallas import tpu as pltpu
from jax.experimental.pallas import tpu_sc as plsc

_N = 10000
_D = 128
_NPAD = 10224
_NW = 32
_CH = 128
_NCH = 80
_EPAD = _NW * _NCH * _CH
_WB = 80
_NPADD = 10240
_NPTD = _NPADD // 16


def _sc_degree(dsts):
    mesh = plsc.VectorSubcoreMesh(core_axis_name="c", subcore_axis_name="s")

    @functools.partial(
        pl.kernel,
        out_type=jax.ShapeDtypeStruct((2, _NPADD), jnp.float32),
        mesh=mesh,
        scratch_types=[
            pltpu.VMEM((_NCH, _CH), jnp.int32),
            pltpu.VMEM((_CH,), jnp.float32),
            pltpu.VMEM((_NPTD,), jnp.float32),
            pltpu.VMEM_SHARED((_NPADD,), jnp.float32),
        ],
    )
    def k(dsts_hbm, out_hbm, dst_v, ones_v, stage_v, deg_sh):
        c = lax.axis_index("c")
        s = lax.axis_index("s")
        wid = s * 2 + c
        for i in range(_CH // 16):
            ones_v[pl.ds(i * 16, 16)] = jnp.ones((16,), jnp.float32)
        for i in range(_NPTD // 16):
            stage_v[pl.ds(i * 16, 16)] = jnp.zeros((16,), jnp.float32)
        pltpu.sync_copy(stage_v, deg_sh.at[pl.ds(s * _NPTD, _NPTD)])
        plsc.subcore_barrier()
        pltpu.sync_copy(dsts_hbm.at[wid], dst_v)

        def chunk(j, carry):
            pltpu.sync_copy(ones_v, deg_sh.at[dst_v.at[j]], add=True)
            return carry

        lax.fori_loop(0, _NCH, chunk, 0)
        plsc.subcore_barrier()
        pltpu.sync_copy(deg_sh.at[pl.ds(s * _NPTD, _NPTD)], stage_v)
        pltpu.sync_copy(stage_v, out_hbm.at[c, pl.ds(s * _NPTD, _NPTD)])

    return k(dsts)


def _sc_edge_agg(hp, srcs, dsts):
    mesh = plsc.VectorSubcoreMesh(core_axis_name="c", subcore_axis_name="s")

    @functools.partial(
        pl.kernel,
        out_type=jax.ShapeDtypeStruct((2, _NPAD, _D), jnp.float32),
        mesh=mesh,
        scratch_types=[
            pltpu.VMEM((_NCH // 2, _CH), jnp.int32),
            pltpu.VMEM((_NCH // 2, _CH), jnp.int32),
            pltpu.VMEM((_CH, _D), jnp.float32),
            pltpu.VMEM((_CH, _D), jnp.float32),
            pltpu.VMEM_SHARED((_NPAD, _D), jnp.float32),
            pltpu.SemaphoreType.DMA,
            pltpu.SemaphoreType.DMA,
            pltpu.SemaphoreType.DMA,
            pltpu.SemaphoreType.DMA,
        ],
    )
    def k(hp_hbm, srcs_hbm, dsts_hbm, out_hbm, src_v, dst_v, rows_a, rows_b,
          acc_sh, sem_ga, sem_gb, sem_sa, sem_sb):
        c = lax.axis_index("c")
        s = lax.axis_index("s")
        wid = s * 2 + c

        def zrow(i, carry):
            for kk in range(_D // 16):
                rows_a[i, pl.ds(kk * 16, 16)] = jnp.zeros((16,), jnp.float32)
            return carry

        lax.fori_loop(0, _WB, zrow, 0)

        @pl.when(s < 15)
        def _():
            for kk in range(8):
                pltpu.sync_copy(rows_a.at[pl.ds(0, _WB)],
                                acc_sh.at[pl.ds(s * 640 + kk * _WB, _WB)])

        @pl.when(s == 15)
        def _():
            for kk in range(7):
                pltpu.sync_copy(rows_a.at[pl.ds(0, _WB)],
                                acc_sh.at[pl.ds(9600 + kk * _WB, _WB)])
            pltpu.sync_copy(rows_a.at[pl.ds(0, 64)],
                            acc_sh.at[pl.ds(10160, 64)])

        plsc.subcore_barrier()

        half = _NCH // 2
        for ph in range(2):
            pltpu.sync_copy(srcs_hbm.at[wid, pl.ds(ph * half, half)], src_v)
            pltpu.sync_copy(dsts_hbm.at[wid, pl.ds(ph * half, half)], dst_v)
            pltpu.async_copy(hp_hbm.at[src_v.at[0]], rows_a, sem_ga)

            def chunk2(t, carry):
                j0 = 2 * t
                pltpu.make_async_copy(hp_hbm.at[src_v.at[j0]], rows_a,
                                      sem_ga).wait()
                pltpu.async_copy(rows_a, acc_sh.at[dst_v.at[j0]], sem_sa,
                                 add=True)

                @pl.when(t > 0)
                def _():
                    pltpu.make_async_copy(rows_b, acc_sh.at[dst_v.at[j0 - 1]],
                                          sem_sb).wait()

                pltpu.async_copy(hp_hbm.at[src_v.at[j0 + 1]], rows_b, sem_gb)
                pltpu.make_async_copy(hp_hbm.at[src_v.at[j0 + 1]], rows_b,
                                      sem_gb).wait()
                pltpu.async_copy(rows_b, acc_sh.at[dst_v.at[j0 + 1]], sem_sb,
                                 add=True)
                pltpu.make_async_copy(rows_a, acc_sh.at[dst_v.at[j0]],
                                      sem_sa).wait()

                @pl.when(t < half // 2 - 1)
                def _():
                    pltpu.async_copy(hp_hbm.at[src_v.at[j0 + 2]], rows_a,
                                     sem_ga)

                return carry

            lax.fori_loop(0, half // 2, chunk2, 0)
            pltpu.make_async_copy(rows_b, acc_sh.at[dst_v.at[half - 1]],
                                  sem_sb).wait()
        plsc.subcore_barrier()

        @pl.when(s < 15)
        def _():
            sl = pl.ds(s * 640, 640)
            pltpu.sync_copy(acc_sh.at[sl], out_hbm.at[c, sl])

        @pl.when(s == 15)
        def _():
            sl = pl.ds(9600, 624)
            pltpu.sync_copy(acc_sh.at[sl], out_hbm.at[c, sl])

    return k(hp, srcs, dsts)


_BR = 1704


def _gelu(x):
    return 0.5 * x * (1.0 + lax.erf(x * (2.0 ** -0.5)))


def _t1_body(x_ref, deg_ref, w_ref, dis_ref, hp_ref):
    deg = deg_ref[0] + deg_ref[1] + 1.0
    dis = lax.rsqrt(deg)
    dis_ref[...] = dis
    hp_ref[...] = dis * jnp.dot(x_ref[...], w_ref[...],
                                preferred_element_type=jnp.float32)


def _tc_first(xp, degs, w1):
    return pl.pallas_call(
        _t1_body,
        grid=(_NPAD // _BR,),
        in_specs=[
            pl.BlockSpec((_BR, _D), lambda i: (i, 0)),
            pl.BlockSpec((2, _BR, 1), lambda i: (0, i, 0)),
            pl.BlockSpec((_D, _D), lambda i: (0, 0)),
        ],
        out_specs=[
            pl.BlockSpec((_BR, 1), lambda i: (i, 0)),
            pl.BlockSpec((_BR, _D), lambda i: (i, 0)),
        ],
        out_shape=[
            jax.ShapeDtypeStruct((_NPAD, 1), jnp.float32),
            jax.ShapeDtypeStruct((_NPAD, _D), jnp.float32),
        ],
    )(xp, degs, w1)


def _tmid_body(acc_ref, hp_ref, dis_ref, b_ref, w_ref, out_ref):
    ssum = acc_ref[0] + acc_ref[1] + hp_ref[...]
    dis = dis_ref[...]
    pre = dis * ssum + b_ref[...]
    xg = _gelu(pre)
    out_ref[...] = dis * jnp.dot(xg, w_ref[...],
                                 preferred_element_type=jnp.float32)


def _tc_mid(acc, hp, dis, b, w):
    return pl.pallas_call(
        _tmid_body,
        grid=(_NPAD // _BR,),
        in_specs=[
            pl.BlockSpec((2, _BR, _D), lambda i: (0, i, 0)),
            pl.BlockSpec((_BR, _D), lambda i: (i, 0)),
            pl.BlockSpec((_BR, 1), lambda i: (i, 0)),
            pl.BlockSpec((1, _D), lambda i: (0, 0)),
            pl.BlockSpec((_D, _D), lambda i: (0, 0)),
        ],
        out_specs=pl.BlockSpec((_BR, _D), lambda i: (i, 0)),
        out_shape=jax.ShapeDtypeStruct((_NPAD, _D), jnp.float32),
    )(acc, hp, dis, b, w)


def _tfin_body(acc_ref, hp_ref, dis_ref, b_ref, out_ref):
    ssum = acc_ref[0] + acc_ref[1] + hp_ref[...]
    out_ref[...] = dis_ref[...] * ssum + b_ref[...]


def _tc_final(acc, hp, dis, b):
    return pl.pallas_call(
        _tfin_body,
        grid=(_NPAD // _BR,),
        in_specs=[
            pl.BlockSpec((2, _BR, _D), lambda i: (0, i, 0)),
            pl.BlockSpec((_BR, _D), lambda i: (i, 0)),
            pl.BlockSpec((_BR, 1), lambda i: (i, 0)),
            pl.BlockSpec((1, _D), lambda i: (0, 0)),
        ],
        out_specs=pl.BlockSpec((_BR, _D), lambda i: (i, 0)),
        out_shape=jax.ShapeDtypeStruct((_NPAD, _D), jnp.float32),
    )(acc, hp, dis, b)


def kernel(x_piece, edge_index_piece, batch, W1, b1, W2, b2, W3, b3):
    del batch
    src = edge_index_piece[0].astype(jnp.int32)
    dst = edge_index_piece[1].astype(jnp.int32)
    e = src.shape[0]
    pad = _EPAD - e
    padidx = _N + (jnp.arange(pad, dtype=jnp.int32) % (_NPAD - _N))
    srcp = jnp.concatenate([src, padidx])
    dstp = jnp.concatenate([dst, padidx])
    srcp = srcp.reshape(_NW, _NCH, _CH)
    dstp = dstp.reshape(_NW, _NCH, _CH)
    xp = jnp.concatenate(
        [x_piece, jnp.zeros((_NPAD - _N, _D), jnp.float32)], axis=0)

    degs = _sc_degree(dstp)[:, :_NPAD].reshape(2, _NPAD, 1)
    dis, hp1 = _tc_first(xp, degs, W1)
    acc1 = _sc_edge_agg(hp1, srcp, dstp)
    hp2 = _tc_mid(acc1, hp1, dis, b1.reshape(1, _D), W2)
    acc2 = _sc_edge_agg(hp2, srcp, dstp)
    hp3 = _tc_mid(acc2, hp2, dis, b2.reshape(1, _D), W3)
    acc3 = _sc_edge_agg(hp3, srcp, dstp)
    out = _tc_final(acc3, hp3, dis, b3.reshape(1, _D))
    return out[:_N]


# speedup vs baseline: 3.7937x; 1.1187x over previous
import functools

import jax
import jax.numpy as jnp
from jax import lax
from jax.experimental import pallas as pl
from jax.experimental.pallas import tpu as pltpu
from jax.experimental.pallas import tpu_sc as plsc

_N = 10000
_D = 128
_NPAD = 10224
_NW = 32
_CH = 64
_NCH = 160
_NPH = 4
_EPAD = _NW * _NCH * _CH
_WB = 80
_NPADD = 10240
_NPTD = _NPADD // 16


def _sc_degree(dsts):
    mesh = plsc.VectorSubcoreMesh(core_axis_name="c", subcore_axis_name="s")

    @functools.partial(
        pl.kernel,
        out_type=jax.ShapeDtypeStruct((2, _NPADD), jnp.float32),
        mesh=mesh,
        scratch_types=[
            pltpu.VMEM((_NCH, _CH), jnp.int32),
            pltpu.VMEM((_CH,), jnp.float32),
            pltpu.VMEM((_NPTD,), jnp.float32),
            pltpu.VMEM_SHARED((_NPADD,), jnp.float32),
        ],
    )
    def k(dsts_hbm, out_hbm, dst_v, ones_v, stage_v, deg_sh):
        c = lax.axis_index("c")
        s = lax.axis_index("s")
        wid = s * 2 + c
        for i in range(_CH // 16):
            ones_v[pl.ds(i * 16, 16)] = jnp.ones((16,), jnp.float32)
        for i in range(_NPTD // 16):
            stage_v[pl.ds(i * 16, 16)] = jnp.zeros((16,), jnp.float32)
        pltpu.sync_copy(stage_v, deg_sh.at[pl.ds(s * _NPTD, _NPTD)])
        plsc.subcore_barrier()
        pltpu.sync_copy(dsts_hbm.at[wid], dst_v)

        def chunk(j, carry):
            pltpu.sync_copy(ones_v, deg_sh.at[dst_v.at[j]], add=True)
            return carry

        lax.fori_loop(0, _NCH, chunk, 0)
        plsc.subcore_barrier()
        pltpu.sync_copy(deg_sh.at[pl.ds(s * _NPTD, _NPTD)], stage_v)
        pltpu.sync_copy(stage_v, out_hbm.at[c, pl.ds(s * _NPTD, _NPTD)])

    return k(dsts)


def _sc_edge_agg(hp, srcs, dsts):
    mesh = plsc.VectorSubcoreMesh(core_axis_name="c", subcore_axis_name="s")

    @functools.partial(
        pl.kernel,
        out_type=jax.ShapeDtypeStruct((2, _NPAD, _D), jnp.float32),
        mesh=mesh,
        scratch_types=[
            pltpu.VMEM((_NCH // _NPH, _CH), jnp.int32),
            pltpu.VMEM((_NCH // _NPH, _CH), jnp.int32),
            pltpu.VMEM((_CH, _D), jnp.float32),
            pltpu.VMEM((_CH, _D), jnp.float32),
            pltpu.VMEM((_CH, _D), jnp.float32),
            pltpu.VMEM((_CH, _D), jnp.float32),
            pltpu.VMEM_SHARED((_NPAD, _D), jnp.float32),
            pltpu.SemaphoreType.DMA,
            pltpu.SemaphoreType.DMA,
            pltpu.SemaphoreType.DMA,
            pltpu.SemaphoreType.DMA,
            pltpu.SemaphoreType.DMA,
            pltpu.SemaphoreType.DMA,
            pltpu.SemaphoreType.DMA,
            pltpu.SemaphoreType.DMA,
        ],
    )
    def k(hp_hbm, srcs_hbm, dsts_hbm, out_hbm, src_v, dst_v,
          rows0, rows1, rows2, rows3, acc_sh,
          sg0, sg1, sg2, sg3, ss0, ss1, ss2, ss3):
        c = lax.axis_index("c")
        s = lax.axis_index("s")
        wid = s * 2 + c
        rows = [rows0, rows1, rows2, rows3]
        sg = [sg0, sg1, sg2, sg3]
        ss = [ss0, ss1, ss2, ss3]

        def zrow(i, carry):
            for kk in range(_D // 16):
                rows0[i, pl.ds(kk * 16, 16)] = jnp.zeros((16,), jnp.float32)
            return carry

        lax.fori_loop(0, _CH, zrow, 0)

        ssl = [ss0, ss1, ss2, ss3]

        @pl.when(s < 15)
        def _():
            for kk in range(10):
                pltpu.async_copy(rows0.at[pl.ds(0, 64)],
                                 acc_sh.at[pl.ds(s * 640 + kk * 64, 64)],
                                 ssl[kk % 4])
            for kk in range(10):
                pltpu.make_async_copy(
                    rows0.at[pl.ds(0, 64)],
                    acc_sh.at[pl.ds(s * 640 + kk * 64, 64)],
                    ssl[kk % 4]).wait()

        @pl.when(s == 15)
        def _():
            for kk in range(9):
                pltpu.async_copy(rows0.at[pl.ds(0, 64)],
                                 acc_sh.at[pl.ds(9600 + kk * 64, 64)],
                                 ssl[kk % 4])
            pltpu.async_copy(rows0.at[pl.ds(0, 48)],
                             acc_sh.at[pl.ds(10176, 48)], ssl[1])
            for kk in range(9):
                pltpu.make_async_copy(
                    rows0.at[pl.ds(0, 64)],
                    acc_sh.at[pl.ds(9600 + kk * 64, 64)],
                    ssl[kk % 4]).wait()
            pltpu.make_async_copy(rows0.at[pl.ds(0, 48)],
                                  acc_sh.at[pl.ds(10176, 48)], ssl[1]).wait()

        plsc.subcore_barrier()

        npp = _NCH // _NPH

        def g_issue(k, j):
            pltpu.async_copy(hp_hbm.at[src_v.at[j]], rows[k], sg[k])

        def g_wait(k, j):
            pltpu.make_async_copy(hp_hbm.at[src_v.at[j]], rows[k],
                                  sg[k]).wait()

        def s_issue(k, j):
            pltpu.async_copy(rows[k], acc_sh.at[dst_v.at[j]], ss[k],
                             add=True)

        def s_wait(k, j):
            pltpu.make_async_copy(rows[k], acc_sh.at[dst_v.at[j]],
                                  ss[k]).wait()

        for ph in range(_NPH):
            pltpu.sync_copy(srcs_hbm.at[wid, pl.ds(ph * npp, npp)], src_v)
            pltpu.sync_copy(dsts_hbm.at[wid, pl.ds(ph * npp, npp)], dst_v)

            def group4(t, carry):
                j0 = 4 * t
                for k in range(4):
                    jk = j0 + k
                    k2 = (k + 2) % 4

                    @pl.when(t > 0)
                    def _(k=k, jk=jk):
                        s_wait(k, jk - 4)

                    g_issue(k, jk)
                    if k >= 2:
                        g_wait(k2, jk - 2)
                        s_issue(k2, jk - 2)
                    else:
                        @pl.when(t > 0)
                        def _(k2=k2, jk=jk):
                            g_wait(k2, jk - 2)
                            s_issue(k2, jk - 2)
                return carry

            lax.fori_loop(0, npp // 4, group4, 0)
            s_wait(0, npp - 4)
            s_wait(1, npp - 3)
            g_wait(2, npp - 2)
            s_issue(2, npp - 2)
            g_wait(3, npp - 1)
            s_issue(3, npp - 1)
            s_wait(2, npp - 2)
            s_wait(3, npp - 1)
        plsc.subcore_barrier()

        @pl.when(s < 15)
        def _():
            sl = pl.ds(s * 640, 640)
            pltpu.sync_copy(acc_sh.at[sl], out_hbm.at[c, sl])

        @pl.when(s == 15)
        def _():
            sl = pl.ds(9600, 624)
            pltpu.sync_copy(acc_sh.at[sl], out_hbm.at[c, sl])

    return k(hp, srcs, dsts)


_BR = 1704


def _gelu(x):
    return 0.5 * x * (1.0 + lax.erf(x * (2.0 ** -0.5)))


def _t1_body(x_ref, deg_ref, w_ref, dis_ref, hp_ref):
    deg = deg_ref[0] + deg_ref[1] + 1.0
    dis = lax.rsqrt(deg)
    dis_ref[...] = dis
    hp_ref[...] = dis * jnp.dot(x_ref[...], w_ref[...],
                                preferred_element_type=jnp.float32)


def _tc_first(xp, degs, w1):
    return pl.pallas_call(
        _t1_body,
        grid=(_NPAD // _BR,),
        in_specs=[
            pl.BlockSpec((_BR, _D), lambda i: (i, 0)),
            pl.BlockSpec((2, _BR, 1), lambda i: (0, i, 0)),
            pl.BlockSpec((_D, _D), lambda i: (0, 0)),
        ],
        out_specs=[
            pl.BlockSpec((_BR, 1), lambda i: (i, 0)),
            pl.BlockSpec((_BR, _D), lambda i: (i, 0)),
        ],
        out_shape=[
            jax.ShapeDtypeStruct((_NPAD, 1), jnp.float32),
            jax.ShapeDtypeStruct((_NPAD, _D), jnp.float32),
        ],
    )(xp, degs, w1)


def _tmid_body(acc_ref, hp_ref, dis_ref, b_ref, w_ref, out_ref):
    ssum = acc_ref[0] + acc_ref[1] + hp_ref[...]
    dis = dis_ref[...]
    pre = dis * ssum + b_ref[...]
    xg = _gelu(pre)
    out_ref[...] = dis * jnp.dot(xg, w_ref[...],
                                 preferred_element_type=jnp.float32)


def _tc_mid(acc, hp, dis, b, w):
    return pl.pallas_call(
        _tmid_body,
        grid=(_NPAD // _BR,),
        in_specs=[
            pl.BlockSpec((2, _BR, _D), lambda i: (0, i, 0)),
            pl.BlockSpec((_BR, _D), lambda i: (i, 0)),
            pl.BlockSpec((_BR, 1), lambda i: (i, 0)),
            pl.BlockSpec((1, _D), lambda i: (0, 0)),
            pl.BlockSpec((_D, _D), lambda i: (0, 0)),
        ],
        out_specs=pl.BlockSpec((_BR, _D), lambda i: (i, 0)),
        out_shape=jax.ShapeDtypeStruct((_NPAD, _D), jnp.float32),
    )(acc, hp, dis, b, w)


def _tfin_body(acc_ref, hp_ref, dis_ref, b_ref, out_ref):
    ssum = acc_ref[0] + acc_ref[1] + hp_ref[...]
    out_ref[...] = dis_ref[...] * ssum + b_ref[...]


def _tc_final(acc, hp, dis, b):
    return pl.pallas_call(
        _tfin_body,
        grid=(_NPAD // _BR,),
        in_specs=[
            pl.BlockSpec((2, _BR, _D), lambda i: (0, i, 0)),
            pl.BlockSpec((_BR, _D), lambda i: (i, 0)),
            pl.BlockSpec((_BR, 1), lambda i: (i, 0)),
            pl.BlockSpec((1, _D), lambda i: (0, 0)),
        ],
        out_specs=pl.BlockSpec((_BR, _D), lambda i: (i, 0)),
        out_shape=jax.ShapeDtypeStruct((_NPAD, _D), jnp.float32),
    )(acc, hp, dis, b)


def kernel(x_piece, edge_index_piece, batch, W1, b1, W2, b2, W3, b3):
    del batch
    src = edge_index_piece[0].astype(jnp.int32)
    dst = edge_index_piece[1].astype(jnp.int32)
    e = src.shape[0]
    pad = _EPAD - e
    padidx = _N + (jnp.arange(pad, dtype=jnp.int32) % (_NPAD - _N))
    srcp = jnp.concatenate([src, padidx])
    dstp = jnp.concatenate([dst, padidx])
    srcp = srcp.reshape(_NW, _NCH, _CH)
    dstp = dstp.reshape(_NW, _NCH, _CH)
    xp = jnp.concatenate(
        [x_piece, jnp.zeros((_NPAD - _N, _D), jnp.float32)], axis=0)

    degs = _sc_degree(dstp)[:, :_NPAD].reshape(2, _NPAD, 1)
    dis, hp1 = _tc_first(xp, degs, W1)
    acc1 = _sc_edge_agg(hp1, srcp, dstp)
    hp2 = _tc_mid(acc1, hp1, dis, b1.reshape(1, _D), W2)
    acc2 = _sc_edge_agg(hp2, srcp, dstp)
    hp3 = _tc_mid(acc2, hp2, dis, b2.reshape(1, _D), W3)
    acc3 = _sc_edge_agg(hp3, srcp, dstp)
    out = _tc_final(acc3, hp3, dis, b3.reshape(1, _D))
    return out[:_N]
